# Initial kernel scaffold; baseline (speedup 1.0000x reference)
import jax
import jax.numpy as jnp
from jax.experimental import pallas as pl


def kernel(x, table, f):
    raise NotImplementedError("write your pallas kernel here")


# speedup vs baseline: 1.3851x; 1.3851x over previous
import functools

import jax
import jax.numpy as jnp
from jax import lax
from jax.experimental import pallas as pl
from jax.experimental.pallas import tpu as pltpu
from jax.experimental.pallas import tpu_sc as plsc

_BATCH = 4096
_MAX_LEN = 200
_EMBED = 64
_NC = 2
_NS = 16
_NW = _NC * _NS
_BW = _BATCH // _NW
_LANES = 16
_NJ = _EMBED // _LANES
_UNROLL = 4


def _make_sc_call():
  mesh = plsc.VectorSubcoreMesh(core_axis_name="c", subcore_axis_name="s")

  @functools.partial(
      pl.kernel,
      mesh=mesh,
      out_type=jax.ShapeDtypeStruct((_BATCH, _EMBED), jnp.float32),
      compiler_params=pltpu.CompilerParams(use_tc_tiling_on_sc=False),
      scratch_types=[
          pltpu.VMEM((_MAX_LEN, _BW), jnp.int32),
          pltpu.VMEM((_MAX_LEN, _EMBED), jnp.float32),
          pltpu.VMEM((_BW, _EMBED), jnp.float32),
          pltpu.VMEM((_BW, _EMBED), jnp.float32),
          pltpu.VMEM((_BW, _EMBED), jnp.float32),
          pltpu.SemaphoreType.DMA,
          pltpu.SemaphoreType.DMA,
      ],
  )
  def sc_encoder(xw_hbm, table_hbm, f_hbm, out_hbm,
                 idx_v, f_v, rows0, rows1, acc_v, sem0, sem1):
    wid = lax.axis_index("s") * _NC + lax.axis_index("c")
    base = wid * _BW

    pltpu.sync_copy(xw_hbm.at[wid], idx_v)
    pltpu.sync_copy(f_hbm, f_v)

    zero = jnp.zeros((_LANES,), jnp.float32)

    def zbody(b, carry):
      for j in range(_NJ):
        acc_v[b, pl.ds(j * _LANES, _LANES)] = zero
      return carry

    lax.fori_loop(0, _BW, zbody, 0)

    idx = idx_v

    def gather(l, rows, sem):
      pltpu.async_copy(table_hbm.at[idx.at[l]], rows, sem)

    def gwait(rows, sem):
      pltpu.make_async_copy(table_hbm.at[idx.at[0]], rows, sem).wait()

    def accum(rows, l):
      fv = [f_v[l, pl.ds(j * _LANES, _LANES)] for j in range(_NJ)]

      def bbody(b, carry):
        for u in range(_UNROLL):
          bb = b * _UNROLL + u
          for j in range(_NJ):
            sl = pl.ds(j * _LANES, _LANES)
            plsc.addupdate(acc_v.at[bb, sl], rows[bb, sl] * fv[j])
        return carry

      lax.fori_loop(0, _BW // _UNROLL, bbody, 0)

    gather(0, rows0, sem0)
    gather(1, rows1, sem1)

    def lbody(i, carry):
      l0 = 2 * i
      gwait(rows0, sem0)
      accum(rows0, l0)
      gather(l0 + 2, rows0, sem0)
      gwait(rows1, sem1)
      accum(rows1, l0 + 1)
      gather(l0 + 3, rows1, sem1)
      return carry

    lax.fori_loop(0, _MAX_LEN // 2 - 1, lbody, 0)

    gwait(rows0, sem0)
    accum(rows0, _MAX_LEN - 2)
    gwait(rows1, sem1)
    accum(rows1, _MAX_LEN - 1)

    pltpu.sync_copy(acc_v, out_hbm.at[pl.ds(base, _BW)])

  return sc_encoder


_sc_encoder = _make_sc_call()


@jax.jit
def kernel(x, table, f):
  xw = x.astype(jnp.int32).T.reshape(_MAX_LEN, _NW, _BW).transpose(1, 0, 2)
  return _sc_encoder(xw, table, f.astype(jnp.float32))
